# baseline (speedup 1.0000x reference)
import functools

import jax
import jax.numpy as jnp
from jax import lax
from jax.experimental import pallas as pl
from jax.experimental.pallas import tpu as pltpu
from jax.experimental.pallas import tpu_sc as plsc

_INFO = plsc.get_sparse_core_info()
_NC, _NS = _INFO.num_cores, _INFO.num_subcores
_NW = _NC * _NS

_CHUNK = 128
_GPM = 2
_NBUF = 3


@functools.partial(jax.jit, static_argnums=(2, 3))
def _lookup(idx3, table, n_chunks, d):
    per_w = n_chunks * _CHUNK
    b_total = _NW * per_w
    mesh = plsc.VectorSubcoreMesh(core_axis_name="c", subcore_axis_name="s")
    n_m = n_chunks // _GPM
    rows_m = _CHUNK * _GPM

    @functools.partial(
        pl.kernel,
        out_type=jax.ShapeDtypeStruct((b_total, d), jnp.float32),
        mesh=mesh,
        scratch_types=[
            pltpu.VMEM((n_chunks, _CHUNK), jnp.int32),
            [pltpu.VMEM((rows_m, d), jnp.float32) for _ in range(_NBUF)],
            [pltpu.SemaphoreType.DMA for _ in range(_NBUF)],
            [pltpu.SemaphoreType.DMA for _ in range(_NBUF)],
        ],
    )
    def ker(idx_hbm, table_hbm, out_hbm, idx_v, bufs, gsems, wsems):
        wid = lax.axis_index("s") * _NC + lax.axis_index("c")
        base = wid * per_w
        pltpu.sync_copy(idx_hbm.at[wid], idx_v)

        def g_copy(m, h, b):
            return pltpu.make_async_copy(
                table_hbm.at[idx_v.at[m * _GPM + h]],
                bufs[b].at[pl.ds(h * _CHUNK, _CHUNK)],
                gsems[b],
            )

        def start_g(m, b):
            for h in range(_GPM):
                g_copy(m, h, b).start()

        def wait_g(m, b):
            for h in range(_GPM):
                g_copy(m, h, b).wait()

        def w_copy(m, b):
            return pltpu.make_async_copy(
                bufs[b], out_hbm.at[pl.ds(base + m * rows_m, rows_m)], wsems[b]
            )

        start_g(0, 0)
        start_g(1, 1)

        def full_iter(m, b):
            wait_g(m, b)
            w_copy(m, b).start()
            w_copy(m - 1, (b - 1) % _NBUF).wait()
            start_g(m + 2, (b + 2) % _NBUF)

        wait_g(0, 0)
        w_copy(0, 0).start()
        start_g(2, 2)
        full_iter(1, 1)
        full_iter(2, 2)

        def body(g, carry):
            for b in range(_NBUF):
                full_iter(g * _NBUF + b, b)
            return carry

        lax.fori_loop(1, (n_m - 4) // _NBUF, body, 0)

        for m in (n_m - 4, n_m - 3):
            b = m % _NBUF
            full_iter(m, b)
        for m in (n_m - 2, n_m - 1):
            b = m % _NBUF
            wait_g(m, b)
            w_copy(m, b).start()
            w_copy(m - 1, (b - 1) % _NBUF).wait()
        w_copy(n_m - 1, (n_m - 1) % _NBUF).wait()

    return ker(idx3, table)


def kernel(inputs, table):
    bsz, seq = inputs.shape
    d = table.shape[1]
    b_total = bsz * seq
    assert b_total % (_NW * _CHUNK * _GPM) == 0
    n_chunks = b_total // (_NW * _CHUNK)
    idx3 = jnp.reshape(inputs, (_NW, n_chunks, _CHUNK))
    out = _lookup(idx3, table, n_chunks, d)
    return jnp.reshape(out, (bsz, seq, d))


# speedup vs baseline: 1.0033x; 1.0033x over previous
import functools

import jax
import jax.numpy as jnp
from jax import lax
from jax.experimental import pallas as pl
from jax.experimental.pallas import tpu as pltpu
from jax.experimental.pallas import tpu_sc as plsc

_INFO = plsc.get_sparse_core_info()
_NC, _NS = _INFO.num_cores, _INFO.num_subcores
_NW = _NC * _NS

_CHUNK = 128
_NBUF = 5
_LAG = 2


@functools.partial(jax.jit, static_argnums=(2, 3))
def _lookup(idx3, table, n_chunks, d):
    per_w = n_chunks * _CHUNK
    b_total = _NW * per_w
    mesh = plsc.VectorSubcoreMesh(core_axis_name="c", subcore_axis_name="s")
    n_groups = n_chunks // _NBUF

    @functools.partial(
        pl.kernel,
        out_type=jax.ShapeDtypeStruct((b_total, d), jnp.float32),
        mesh=mesh,
        scratch_types=[
            pltpu.VMEM((n_chunks, _CHUNK), jnp.int32),
            [pltpu.VMEM((_CHUNK, d), jnp.float32) for _ in range(_NBUF)],
            [pltpu.SemaphoreType.DMA for _ in range(_NBUF)],
            [pltpu.SemaphoreType.DMA for _ in range(_NBUF)],
        ],
    )
    def ker(idx_hbm, table_hbm, out_hbm, idx_v, bufs, gsems, wsems):
        wid = lax.axis_index("s") * _NC + lax.axis_index("c")
        base = wid * per_w
        pltpu.sync_copy(idx_hbm.at[wid], idx_v)

        def start_g(j, b):
            pltpu.async_copy(table_hbm.at[idx_v.at[j]], bufs[b], gsems[b])

        def wait_g(j, b):
            pltpu.make_async_copy(
                table_hbm.at[idx_v.at[j]], bufs[b], gsems[b]
            ).wait()

        def out_slice(j):
            return out_hbm.at[pl.ds(base + j * _CHUNK, _CHUNK)]

        def start_w(j, b):
            pltpu.async_copy(bufs[b], out_slice(j), wsems[b])

        def wait_w(j, b):
            pltpu.make_async_copy(bufs[b], out_slice(j), wsems[b]).wait()

        prime = _NBUF - _LAG
        n = n_chunks

        for jj in range(prime):
            start_g(jj, jj)

        def full_iter(j, b):
            wait_g(j, b)
            start_w(j, b)
            wait_w(j - _LAG, (b - _LAG) % _NBUF)
            start_g(j + prime, (b + prime) % _NBUF)

        for b in range(_NBUF):
            j = b
            wait_g(j, b)
            start_w(j, b)
            if j >= _LAG:
                wait_w(j - _LAG, (b - _LAG) % _NBUF)
            start_g(j + prime, (b + prime) % _NBUF)

        def body(g, carry):
            for b in range(_NBUF):
                full_iter(g * _NBUF + b, b)
            return carry

        lax.fori_loop(1, n_groups - 1, body, 0)

        for b in range(_NBUF):
            j = (n_groups - 1) * _NBUF + b
            wait_g(j, b)
            start_w(j, b)
            wait_w(j - _LAG, (b - _LAG) % _NBUF)
            if j + prime < n:
                start_g(j + prime, (b + prime) % _NBUF)
        for j in range(n - _LAG, n):
            wait_w(j, j % _NBUF)

    return ker(idx3, table)


def kernel(inputs, table):
    bsz, seq = inputs.shape
    d = table.shape[1]
    b_total = bsz * seq
    assert b_total % (_NW * _CHUNK) == 0
    n_chunks = b_total // (_NW * _CHUNK)
    idx3 = jnp.reshape(inputs, (_NW, n_chunks, _CHUNK))
    out = _lookup(idx3, table, n_chunks, d)
    return jnp.reshape(out, (bsz, seq, d))
